# Initial kernel scaffold; baseline (speedup 1.0000x reference)
import jax
import jax.numpy as jnp
from jax.experimental import pallas as pl


def kernel(proposals, scores, nms_thres, nms_topk):
    raise NotImplementedError("write your pallas kernel here")


# speedup vs baseline: 20.4483x; 20.4483x over previous
import jax
import jax.numpy as jnp
from jax.experimental import pallas as pl
from jax.experimental.pallas import tpu as pltpu

_N_OFFSETS = 72
_N_STRIPS = _N_OFFSETS - 1
_P = 1024
_L = 128


def _nms_kernel(xs_ref, xst_ref, st_ref, en_ref, stt_ref, ent_ref, t_ref,
                keep_ref, B_ref, v_ref, vt_ref, n_rows: int):
    kio = jax.lax.broadcasted_iota(jnp.int32, (_P, _L), 1).astype(jnp.float32)
    st = st_ref[:, :]
    en = en_ref[:, :]
    v_ref[:, :] = jnp.where((kio >= st) & (kio <= en), 1.0, 0.0)

    kio_t = jax.lax.broadcasted_iota(jnp.int32, (_L, _P), 0).astype(jnp.float32)
    stt = stt_ref[:, :]
    ent = ent_ref[:, :]
    vt_ref[:, :] = jnp.where((kio_t >= stt) & (kio_t <= ent), 1.0, 0.0)

    B_ref[:, :] = jnp.zeros((_P, _P), jnp.float32)
    for k in range(_N_OFFSETS):
        vc = v_ref[:, k:k + 1]
        vr = vt_ref[k:k + 1, :]
        uc = xs_ref[:, k:k + 1] * vc
        ur = xst_ref[k:k + 1, :] * vr
        B_ref[:, :] += jnp.abs(uc * vr - vc * ur)

    t = t_ref[0, 0]
    for rb in range(_P // 128):
        rows = slice(rb * 128, (rb + 1) * 128)
        cnt = jnp.dot(v_ref[rows, :], vt_ref[:, :],
                      preferred_element_type=jnp.float32)
        dsum = B_ref[rows, :]
        dist = jnp.where(cnt > 0, dsum / jnp.maximum(cnt, 1.0), jnp.inf)
        B_ref[rows, :] = jnp.where(dist < t, 1.0, 0.0)

    lane = jax.lax.broadcasted_iota(jnp.int32, (1, _P), 1).astype(jnp.float32)

    def body(i, keep):
        fi = i.astype(jnp.float32)
        keep_i = jnp.sum(jnp.where(lane == fi, keep, 0.0))
        row = B_ref[pl.ds(i, 1), :]
        sup = row * jnp.where(lane > fi, 1.0, 0.0)
        return keep * (1.0 - keep_i * sup)

    keep = jax.lax.fori_loop(0, n_rows, body, jnp.ones((1, _P), jnp.float32))
    keep_ref[:, :] = keep


def kernel(proposals, scores, nms_thres, nms_topk):
    N = proposals.shape[0]
    order = jnp.argsort(-scores)
    ps = proposals[order]

    starts = jnp.clip(jnp.round(ps[:, 2] * _N_STRIPS).astype(jnp.int32),
                      0, _N_STRIPS)
    lengths = jnp.clip(jnp.round(ps[:, 4]).astype(jnp.int32), 1, _N_OFFSETS)
    ends = jnp.clip(starts + lengths - 1, 0, _N_STRIPS)
    xs = ps[:, 5:5 + _N_OFFSETS]

    xs_p = jnp.zeros((_P, _L), jnp.float32).at[:N, :_N_OFFSETS].set(xs)
    st_p = jnp.full((_P, 1), 1e9, jnp.float32).at[:N, 0].set(
        starts.astype(jnp.float32))
    en_p = jnp.full((_P, 1), -1e9, jnp.float32).at[:N, 0].set(
        ends.astype(jnp.float32))
    t = jnp.full((1, 1), nms_thres, jnp.float32)

    import functools
    keep = pl.pallas_call(
        functools.partial(_nms_kernel, n_rows=N),
        out_shape=jax.ShapeDtypeStruct((1, _P), jnp.float32),
        scratch_shapes=[
            pltpu.VMEM((_P, _P), jnp.float32),
            pltpu.VMEM((_P, _L), jnp.float32),
            pltpu.VMEM((_L, _P), jnp.float32),
        ],
    )(xs_p, xs_p.T, st_p, en_p, st_p.T, en_p.T, t)

    keep_sorted = keep[0, :N] > 0.5
    kept_scores_sorted = jnp.where(keep_sorted, scores[order], -jnp.inf)
    top_vals, top_pos = jax.lax.top_k(kept_scores_sorted, 100)
    top_idx = order[top_pos]
    num_kept = jnp.minimum(keep_sorted.sum(), nms_topk)
    return proposals[top_idx], top_vals, top_idx, num_kept
